# baseline (speedup 1.0000x reference)
import functools

import jax
import jax.numpy as jnp
import numpy as np
from jax import lax
from jax.experimental import pallas as pl
from jax.experimental.pallas import tpu as pltpu
from jax.experimental.pallas import tpu_sc as plsc

F = 26
V = 100001
D = 32
B = 16384
P = F * (F - 1) // 2

_GW = 128
_BB = 256
_VCH = 12800
_NVCH = 8
_Q = _VCH // 4
_FH = 13
_NROWH = _FH * _NVCH * _VCH
_NIDXH = B * _FH


def _merge_body(in_ref, eyes_ref, out_ref):
    x = in_ref[0]

    def tr(j):
        return jax.lax.dot_general(
            x[:, j * _Q:(j + 1) * _Q], eyes_ref[:, j * 128:(j + 1) * 128],
            dimension_numbers=(((0,), (0,)), ((), ())),
            preferred_element_type=jnp.float32)

    out_ref[...] = (tr(0) + tr(1)) + (tr(2) + tr(3))


def _merge_tables(tfd, f0):
    eyes = np.zeros((D, 4 * 128), dtype=np.float32)
    for j in range(4):
        eyes[np.arange(D), j * 128 + j * D + np.arange(D)] = 1.0
    return pl.pallas_call(
        _merge_body,
        grid=(_FH, _NVCH),
        in_specs=[pl.BlockSpec((1, D, _VCH), lambda f, k: (f0 + f, 0, k)),
                  pl.BlockSpec((D, 4 * 128), lambda f, k: (0, 0))],
        out_specs=pl.BlockSpec((_Q, 4 * D), lambda f, k: (f * _NVCH + k, 0)),
        out_shape=jax.ShapeDtypeStruct((_NROWH // 4, 4 * D), jnp.float32),
    )(tfd, jnp.asarray(eyes))


def _sc_gather(flat_tables, flat_idx2):
    mesh = plsc.VectorSubcoreMesh(core_axis_name="core",
                                  subcore_axis_name="subcore")

    @functools.partial(
        pl.kernel,
        out_type=jax.ShapeDtypeStruct((_NIDXH, D), jnp.float32),
        mesh=mesh,
        compiler_params=pltpu.CompilerParams(use_tc_tiling_on_sc=False),
    )
    def sc_kernel(tab_hbm, idx_hbm, out_hbm):
        def body(i_vmem, o_vmem):
            pltpu.sync_copy(tab_hbm.at[i_vmem.at[0]], o_vmem)

        pltpu.emit_pipeline(
            body,
            grid=(_NIDXH // _GW,),
            in_specs=[pl.BlockSpec((1, _GW), index_map=lambda i: (0, i))],
            out_specs=[pl.BlockSpec((_GW, D), index_map=lambda i: (i, 0))],
            core_axis_name=("core", "subcore"),
            dimension_semantics=(pltpu.PARALLEL,),
        )(idx_hbm, out_hbm)

    return sc_kernel(flat_tables, flat_idx2)


def _tc_body(ea_ref, eb_ref, a_ref, b1_ref, w2_ref, b2_ref, w3_ref, b3_ref,
             wo_ref, bo_ref, out_ref):
    e = jnp.concatenate([ea_ref[...], eb_ref[...]], axis=1)
    inter = lax.dot_general(
        e, e,
        dimension_numbers=(((2,), (2,)), ((0,), (0,))),
        preferred_element_type=jnp.float32,
    )
    x = inter.reshape(_BB, F * F)
    h = jnp.maximum(x @ a_ref[...] + b1_ref[...], 0.0)
    h = jnp.maximum(h @ w2_ref[...] + b2_ref[...], 0.0)
    h = jnp.maximum(h @ w3_ref[...] + b3_ref[...], 0.0)
    out_ref[...] = h @ wo_ref[...] + bo_ref[...]


def _local_rows(idx_half):
    c = idx_half // _VCH
    off = idx_half - c * _VCH
    j = off // _Q
    m = off - j * _Q
    floc = jnp.arange(_FH, dtype=idx_half.dtype)[None, :]
    r = floc * (_NVCH * _VCH) + c * _VCH + m * 4 + j
    return r.reshape(1, _NIDXH)


def kernel(indices, tables, W1, b1, W2, b2, W3, b3, Wo, bo):
    tfd = jnp.transpose(tables, (0, 2, 1))

    tab_a = _merge_tables(tfd, 0).reshape(_NROWH, D)
    idx_a = _local_rows(indices[:, :_FH])
    emb_a = _sc_gather(tab_a, idx_a)
    tab_b = _merge_tables(tfd, _FH).reshape(_NROWH, D)
    idx_b = _local_rows(indices[:, _FH:])
    emb_b = _sc_gather(tab_b, idx_b)

    e3a = emb_a.reshape(B, _FH, D)
    e3b = emb_b.reshape(B, _FH, D)

    ii, jj = np.tril_indices(F, k=-1)
    tril_pos = jnp.asarray(ii * F + jj, dtype=jnp.int32)
    u1 = W1.shape[1]
    A = jnp.zeros((F * F, u1), dtype=jnp.float32).at[tril_pos].set(W1)

    u2, u3 = W2.shape[1], W3.shape[1]
    grid = (B // _BB,)
    out = pl.pallas_call(
        _tc_body,
        grid=grid,
        in_specs=[
            pl.BlockSpec((_BB, _FH, D), lambda i: (i, 0, 0)),
            pl.BlockSpec((_BB, _FH, D), lambda i: (i, 0, 0)),
            pl.BlockSpec((F * F, u1), lambda i: (0, 0)),
            pl.BlockSpec((1, u1), lambda i: (0, 0)),
            pl.BlockSpec((u1, u2), lambda i: (0, 0)),
            pl.BlockSpec((1, u2), lambda i: (0, 0)),
            pl.BlockSpec((u2, u3), lambda i: (0, 0)),
            pl.BlockSpec((1, u3), lambda i: (0, 0)),
            pl.BlockSpec((u3, 1), lambda i: (0, 0)),
            pl.BlockSpec((1, 1), lambda i: (0, 0)),
        ],
        out_specs=pl.BlockSpec((_BB, 1), lambda i: (i, 0)),
        out_shape=jax.ShapeDtypeStruct((B, 1), jnp.float32),
    )(e3a, e3b, A, b1.reshape(1, u1), W2, b2.reshape(1, u2), W3,
      b3.reshape(1, u3), Wo, bo.reshape(1, 1))
    return out


# speedup vs baseline: 1.2895x; 1.2895x over previous
import functools

import jax
import jax.numpy as jnp
import numpy as np
from jax import lax
from jax.experimental import pallas as pl
from jax.experimental.pallas import tpu as pltpu
from jax.experimental.pallas import tpu_sc as plsc

F = 26
V = 100001
D = 32
B = 16384
P = F * (F - 1) // 2

_GW = 128
_BB = 256
_VCH = 6400
_NVCH = 16
_GSTR = _NVCH * _VCH
_FA = 16
_FB = 10
_NGA = 4
_NGB = 3


def _merge_body(in_ref, eye_ref, out_ref):
    x = in_ref[...].reshape(4 * D, _VCH)
    out_ref[...] = jax.lax.dot_general(
        x, eye_ref[...],
        dimension_numbers=(((0,), (0,)), ((), ())),
        preferred_element_type=jnp.float32)


def _merge_tables(tfd, g0, ngroups):
    return pl.pallas_call(
        _merge_body,
        grid=(ngroups, _NVCH),
        in_specs=[pl.BlockSpec((4, D, _VCH), lambda g, k: (g0 + g, 0, k)),
                  pl.BlockSpec((4 * D, 4 * D), lambda g, k: (0, 0))],
        out_specs=pl.BlockSpec((_VCH, 4 * D), lambda g, k: (g * _NVCH + k, 0)),
        out_shape=jax.ShapeDtypeStruct((ngroups * _GSTR, 4 * D), jnp.float32),
    )(tfd, jnp.asarray(np.eye(4 * D, dtype=np.float32)))


def _sc_gather(flat_tables, flat_idx2, nidx):
    mesh = plsc.VectorSubcoreMesh(core_axis_name="core",
                                  subcore_axis_name="subcore")

    @functools.partial(
        pl.kernel,
        out_type=jax.ShapeDtypeStruct((nidx, D), jnp.float32),
        mesh=mesh,
        compiler_params=pltpu.CompilerParams(use_tc_tiling_on_sc=False),
    )
    def sc_kernel(tab_hbm, idx_hbm, out_hbm):
        def body(i_vmem, o_vmem):
            pltpu.sync_copy(tab_hbm.at[i_vmem.at[0]], o_vmem)

        pltpu.emit_pipeline(
            body,
            grid=(nidx // _GW,),
            in_specs=[pl.BlockSpec((1, _GW), index_map=lambda i: (0, i))],
            out_specs=[pl.BlockSpec((_GW, D), index_map=lambda i: (i, 0))],
            core_axis_name=("core", "subcore"),
            dimension_semantics=(pltpu.PARALLEL,),
        )(idx_hbm, out_hbm)

    return sc_kernel(flat_tables, flat_idx2)


def _tc_body(ea_ref, eb_ref, a_ref, b1_ref, w2_ref, b2_ref, w3_ref, b3_ref,
             wo_ref, bo_ref, out_ref):
    e = jnp.concatenate([ea_ref[...], eb_ref[...]], axis=1)
    inter = lax.dot_general(
        e, e,
        dimension_numbers=(((2,), (2,)), ((0,), (0,))),
        preferred_element_type=jnp.float32,
    )
    x = inter.reshape(_BB, F * F)
    h = jnp.maximum(x @ a_ref[...] + b1_ref[...], 0.0)
    h = jnp.maximum(h @ w2_ref[...] + b2_ref[...], 0.0)
    h = jnp.maximum(h @ w3_ref[...] + b3_ref[...], 0.0)
    out_ref[...] = h @ wo_ref[...] + bo_ref[...]


def _local_rows(idx_half):
    nf = idx_half.shape[1]
    floc = jnp.arange(nf, dtype=idx_half.dtype)[None, :]
    r = ((floc // 4) * _GSTR + idx_half) * 4 + (floc % 4)
    return r.reshape(1, B * nf)


def kernel(indices, tables, W1, b1, W2, b2, W3, b3, Wo, bo):
    tfd = jnp.transpose(tables, (0, 2, 1))

    tab_a = _merge_tables(tfd, 0, _NGA).reshape(_NGA * _GSTR * 4, D)
    idx_a = _local_rows(indices[:, :_FA])
    emb_a = _sc_gather(tab_a, idx_a, B * _FA)
    tab_b = _merge_tables(tfd, _NGA, _NGB).reshape(_NGB * _GSTR * 4, D)
    idx_b = _local_rows(indices[:, _FA:])
    emb_b = _sc_gather(tab_b, idx_b, B * _FB)

    e3a = emb_a.reshape(B, _FA, D)
    e3b = emb_b.reshape(B, _FB, D)

    ii, jj = np.tril_indices(F, k=-1)
    tril_pos = jnp.asarray(ii * F + jj, dtype=jnp.int32)
    u1 = W1.shape[1]
    A = jnp.zeros((F * F, u1), dtype=jnp.float32).at[tril_pos].set(W1)

    u2, u3 = W2.shape[1], W3.shape[1]
    grid = (B // _BB,)
    out = pl.pallas_call(
        _tc_body,
        grid=grid,
        in_specs=[
            pl.BlockSpec((_BB, _FA, D), lambda i: (i, 0, 0)),
            pl.BlockSpec((_BB, _FB, D), lambda i: (i, 0, 0)),
            pl.BlockSpec((F * F, u1), lambda i: (0, 0)),
            pl.BlockSpec((1, u1), lambda i: (0, 0)),
            pl.BlockSpec((u1, u2), lambda i: (0, 0)),
            pl.BlockSpec((1, u2), lambda i: (0, 0)),
            pl.BlockSpec((u2, u3), lambda i: (0, 0)),
            pl.BlockSpec((1, u3), lambda i: (0, 0)),
            pl.BlockSpec((u3, 1), lambda i: (0, 0)),
            pl.BlockSpec((1, 1), lambda i: (0, 0)),
        ],
        out_specs=pl.BlockSpec((_BB, 1), lambda i: (i, 0)),
        out_shape=jax.ShapeDtypeStruct((B, 1), jnp.float32),
    )(e3a, e3b, A, b1.reshape(1, u1), W2, b2.reshape(1, u2), W3,
      b3.reshape(1, u3), Wo, bo.reshape(1, 1))
    return out


# speedup vs baseline: 1.3464x; 1.0441x over previous
import functools

import jax
import jax.numpy as jnp
import numpy as np
from jax import lax
from jax.experimental import pallas as pl
from jax.experimental.pallas import tpu as pltpu
from jax.experimental.pallas import tpu_sc as plsc

F = 26
V = 100001
D = 32
B = 16384
P = F * (F - 1) // 2

_GW = 128
_BB = 512
_VCH = 6400
_NVCH = 16
_GSTR = _NVCH * _VCH
_FA = 16
_FB = 10
_NGA = 4
_NGB = 3


def _merge_body(in_ref, eye_ref, out_ref):
    x = in_ref[...].reshape(4 * D, _VCH)
    out_ref[...] = jax.lax.dot_general(
        x, eye_ref[...],
        dimension_numbers=(((0,), (0,)), ((), ())),
        preferred_element_type=jnp.float32)


def _merge_tables(tfd, g0, ngroups):
    return pl.pallas_call(
        _merge_body,
        grid=(ngroups, _NVCH),
        in_specs=[pl.BlockSpec((4, D, _VCH), lambda g, k: (g0 + g, 0, k)),
                  pl.BlockSpec((4 * D, 4 * D), lambda g, k: (0, 0))],
        out_specs=pl.BlockSpec((_VCH, 4 * D), lambda g, k: (g * _NVCH + k, 0)),
        out_shape=jax.ShapeDtypeStruct((ngroups * _GSTR, 4 * D), jnp.float32),
    )(tfd, jnp.asarray(np.eye(4 * D, dtype=np.float32)))


def _sc_gather(flat_tables, flat_idx2, nidx):
    mesh = plsc.VectorSubcoreMesh(core_axis_name="core",
                                  subcore_axis_name="subcore")

    @functools.partial(
        pl.kernel,
        out_type=jax.ShapeDtypeStruct((nidx, D), jnp.float32),
        mesh=mesh,
        compiler_params=pltpu.CompilerParams(use_tc_tiling_on_sc=False),
    )
    def sc_kernel(tab_hbm, idx_hbm, out_hbm):
        def body(i_vmem, o_vmem):
            pltpu.sync_copy(tab_hbm.at[i_vmem.at[0]], o_vmem)

        pltpu.emit_pipeline(
            body,
            grid=(nidx // _GW,),
            in_specs=[pl.BlockSpec((1, _GW), index_map=lambda i: (0, i))],
            out_specs=[pl.BlockSpec((_GW, D), index_map=lambda i: (i, 0))],
            core_axis_name=("core", "subcore"),
            dimension_semantics=(pltpu.PARALLEL,),
        )(idx_hbm, out_hbm)

    return sc_kernel(flat_tables, flat_idx2)


def _tc_body(ea_ref, eb_ref, a_ref, b1_ref, w2_ref, b2_ref, w3_ref, b3_ref,
             wo_ref, bo_ref, out_ref):
    e = jnp.concatenate([ea_ref[...], eb_ref[...]], axis=1)
    inter = lax.dot_general(
        e, e,
        dimension_numbers=(((2,), (2,)), ((0,), (0,))),
        preferred_element_type=jnp.float32,
    )
    x = inter.reshape(_BB, F * F)
    h = jnp.maximum(x @ a_ref[...] + b1_ref[...], 0.0)
    h = jnp.maximum(h @ w2_ref[...] + b2_ref[...], 0.0)
    h = jnp.maximum(h @ w3_ref[...] + b3_ref[...], 0.0)
    out_ref[...] = h @ wo_ref[...] + bo_ref[...]


def _local_rows(idx_half):
    nf = idx_half.shape[1]
    floc = jnp.arange(nf, dtype=idx_half.dtype)[None, :]
    r = ((floc // 4) * _GSTR + idx_half) * 4 + (floc % 4)
    return r.reshape(1, B * nf)


def kernel(indices, tables, W1, b1, W2, b2, W3, b3, Wo, bo):
    tfd = jnp.transpose(tables, (0, 2, 1))

    tab_a = _merge_tables(tfd, 0, _NGA).reshape(_NGA * _GSTR * 4, D)
    idx_a = _local_rows(indices[:, :_FA])
    emb_a = _sc_gather(tab_a, idx_a, B * _FA)
    tab_b = _merge_tables(tfd, _NGA, _NGB).reshape(_NGB * _GSTR * 4, D)
    idx_b = _local_rows(indices[:, _FA:])
    emb_b = _sc_gather(tab_b, idx_b, B * _FB)

    e3a = emb_a.reshape(B, _FA, D)
    e3b = emb_b.reshape(B, _FB, D)

    ii, jj = np.tril_indices(F, k=-1)
    tril_pos = jnp.asarray(ii * F + jj, dtype=jnp.int32)
    u1 = W1.shape[1]
    A = jnp.zeros((F * F, u1), dtype=jnp.float32).at[tril_pos].set(W1)

    u2, u3 = W2.shape[1], W3.shape[1]
    grid = (B // _BB,)
    out = pl.pallas_call(
        _tc_body,
        grid=grid,
        in_specs=[
            pl.BlockSpec((_BB, _FA, D), lambda i: (i, 0, 0)),
            pl.BlockSpec((_BB, _FB, D), lambda i: (i, 0, 0)),
            pl.BlockSpec((F * F, u1), lambda i: (0, 0)),
            pl.BlockSpec((1, u1), lambda i: (0, 0)),
            pl.BlockSpec((u1, u2), lambda i: (0, 0)),
            pl.BlockSpec((1, u2), lambda i: (0, 0)),
            pl.BlockSpec((u2, u3), lambda i: (0, 0)),
            pl.BlockSpec((1, u3), lambda i: (0, 0)),
            pl.BlockSpec((u3, 1), lambda i: (0, 0)),
            pl.BlockSpec((1, 1), lambda i: (0, 0)),
        ],
        out_specs=pl.BlockSpec((_BB, 1), lambda i: (i, 0)),
        out_shape=jax.ShapeDtypeStruct((B, 1), jnp.float32),
    )(e3a, e3b, A, b1.reshape(1, u1), W2, b2.reshape(1, u2), W3,
      b3.reshape(1, u3), Wo, bo.reshape(1, 1))
    return out
